# baseline (speedup 1.0000x reference)
import functools
import jax
import jax.numpy as jnp
from jax import lax
from jax.experimental import pallas as pl
from jax.experimental.pallas import tpu as pltpu
from jax.experimental.pallas import tpu_sc as plsc

NU = 50000
NI = 50000
E = 800000
D = 64
LAYERS = 3

HALF = 25000
ACC = 25088
RP = ACC // 16
K = 128
INNER = 25
GW = INNER * K
SPAN = 16 * GW
MAXG = -(-E // SPAN)
CAPW = MAXG * SPAN

_mesh = plsc.VectorSubcoreMesh(core_axis_name="c", subcore_axis_name="s")


@functools.partial(
    pl.kernel,
    out_type=jax.ShapeDtypeStruct((2 * ACC, D), jnp.float32),
    mesh=_mesh,
    compiler_params=pltpu.CompilerParams(use_tc_tiling_on_sc=False),
    scratch_types=[
        pltpu.VMEM((16,), jnp.int32),
        pltpu.VMEM((GW,), jnp.int32),
        pltpu.VMEM((GW,), jnp.int32),
        pltpu.VMEM((K,), jnp.int32),
        pltpu.VMEM((K,), jnp.int32),
        pltpu.VMEM((K, D), jnp.float32),
        pltpu.VMEM((K, D), jnp.float32),
        pltpu.VMEM_SHARED((ACC, D), jnp.float32),
        pltpu.SemaphoreType.DMA,
        pltpu.SemaphoreType.DMA,
    ],
)
def _spmm(tab, gidx, dloc, cnts, zeros, out,
          cnt_v, gv, dv, dst0, dst1, rows0, rows1, acc, sem0, sem1):
    c = lax.axis_index("c")
    s = lax.axis_index("s")
    base_row = s * RP

    pltpu.sync_copy(zeros, acc.at[pl.ds(base_row, RP)])
    pltpu.sync_copy(cnts, cnt_v)
    cnt_vec = cnt_v[...]
    ng = jnp.where(c == 0, cnt_vec[0], cnt_vec[1])
    plsc.subcore_barrier()

    def gather_start(j, rows, sem):
        pltpu.async_copy(tab.at[gv.at[pl.ds(j * K, K)]], rows, sem)

    def gather_wait(j, rows, sem):
        pltpu.make_async_copy(tab.at[gv.at[pl.ds(j * K, K)]], rows,
                              sem).wait()

    def scatter(j, rows, dst):
        for m in range(K // 16):
            dst[pl.ds(m * 16, 16)] = dv[pl.ds(j * K + m * 16, 16)]
        pltpu.sync_copy(rows, acc.at[dst], add=True)

    def outer(o, carry):
        off = c * CAPW + (s * ng + o) * GW
        pltpu.sync_copy(gidx.at[pl.ds(off, GW)], gv)
        pltpu.sync_copy(dloc.at[pl.ds(off, GW)], dv)
        gather_start(0, rows0, sem0)

        def pair(p, carry2):
            j0 = 2 * p
            gather_start(j0 + 1, rows1, sem1)
            gather_wait(j0, rows0, sem0)
            scatter(j0, rows0, dst0)
            gather_start(j0 + 2, rows0, sem0)
            gather_wait(j0 + 1, rows1, sem1)
            scatter(j0 + 1, rows1, dst1)
            return carry2

        lax.fori_loop(0, (INNER - 1) // 2, pair, 0)
        gather_wait(INNER - 1, rows0, sem0)
        scatter(INNER - 1, rows0, dst0)
        return carry

    lax.fori_loop(0, ng, outer, 0)
    plsc.subcore_barrier()
    pltpu.sync_copy(acc.at[pl.ds(base_row, RP)],
                    out.at[pl.ds(c * ACC + base_row, RP)])


def _build_part(gather_idx, dst_idx):
    in0 = dst_idx < HALF
    c0 = jnp.cumsum(in0.astype(jnp.int32))
    c1 = jnp.cumsum(jnp.logical_not(in0).astype(jnp.int32))
    n0 = c0[-1]
    n1 = c1[-1]
    pos = jnp.where(in0, c0 - 1, CAPW + c1 - 1)
    local = jnp.where(in0, dst_idx, dst_idx - HALF)
    gfull = jnp.zeros((2 * CAPW,), jnp.int32).at[pos].set(gather_idx)
    dfull = jnp.full((2 * CAPW,), HALF, jnp.int32).at[pos].set(local)
    ng0 = -(-n0 // SPAN)
    ng1 = -(-n1 // SPAN)
    cnts = jnp.zeros((16,), jnp.int32).at[0].set(ng0).at[1].set(ng1)
    return gfull, dfull, cnts


def _unpad(padded):
    return jnp.concatenate([padded[:HALF], padded[ACC:ACC + HALF]], axis=0)


def kernel(users, items, items_neg, edge_users, edge_items,
           user_embeds, item_embeds):
    eu = edge_users.astype(jnp.int32)
    ei = edge_items.astype(jnp.int32)

    u_deg = jnp.bincount(eu, length=NU)
    i_deg = jnp.bincount(ei, length=NI)
    u_norm = jnp.clip(u_deg, 1, None).astype(jnp.float32) ** -0.5
    i_norm = jnp.clip(i_deg, 1, None).astype(jnp.float32) ** -0.5

    gu, du, cu = _build_part(ei, eu)
    gi, di, ci = _build_part(eu, ei)
    zeros = jnp.zeros((RP, D), jnp.float32)

    ue = [user_embeds]
    ie = [item_embeds]
    for _ in range(LAYERS):
        nu = u_norm[:, None] * _unpad(_spmm(i_norm[:, None] * ie[-1],
                                            gu, du, cu, zeros))
        ni = i_norm[:, None] * _unpad(_spmm(u_norm[:, None] * ue[-1],
                                            gi, di, ci, zeros))
        ue.append(nu)
        ie.append(ni)

    final_u = sum(ue) / float(len(ue))
    final_i = sum(ie) / float(len(ie))

    u = final_u[users]
    it = final_i[items]
    it_neg = final_i[items_neg]
    pos = (u * it).sum(-1)
    neg = (u[:, None] * it_neg).sum(-1)
    return pos, neg


# speedup vs baseline: 2.0729x; 2.0729x over previous
import functools
import jax
import jax.numpy as jnp
from jax import lax
from jax.experimental import pallas as pl
from jax.experimental.pallas import tpu as pltpu
from jax.experimental.pallas import tpu_sc as plsc

NU = 50000
NI = 50000
E = 800000
D = 64
LAYERS = 3

HALF = 25000
ACC = 25088
RP = ACC // 16
K = 128
INNER = 25
GW = INNER * K
NG = -(-E // (16 * GW))
TECW = NG * GW
CAPW = 16 * TECW
PAD = CAPW - E

_mesh = plsc.VectorSubcoreMesh(core_axis_name="c", subcore_axis_name="s")


@functools.partial(
    pl.kernel,
    out_type=jax.ShapeDtypeStruct((2 * ACC, D), jnp.float32),
    mesh=_mesh,
    compiler_params=pltpu.CompilerParams(use_tc_tiling_on_sc=False),
    scratch_types=[
        pltpu.VMEM((GW,), jnp.int32),
        pltpu.VMEM((GW,), jnp.int32),
        pltpu.VMEM((K,), jnp.int32),
        pltpu.VMEM((K,), jnp.int32),
        pltpu.VMEM((K, D), jnp.float32),
        pltpu.VMEM((K, D), jnp.float32),
        pltpu.VMEM_SHARED((ACC, D), jnp.float32),
        pltpu.SemaphoreType.DMA,
        pltpu.SemaphoreType.DMA,
    ],
)
def _spmm(tab, gidx, dloc, zeros, out,
          gv, dv, dst0, dst1, rows0, rows1, acc, sem0, sem1):
    c = lax.axis_index("c")
    s = lax.axis_index("s")
    base_row = s * RP

    pltpu.sync_copy(zeros, acc.at[pl.ds(base_row, RP)])
    plsc.subcore_barrier()

    def gather_start(j, rows, sem):
        pltpu.async_copy(tab.at[gv.at[pl.ds(j * K, K)]], rows, sem)

    def gather_wait(j, rows, sem):
        pltpu.make_async_copy(tab.at[gv.at[pl.ds(j * K, K)]], rows,
                              sem).wait()

    def scatter(j, rows, dst):
        for m in range(K // 16):
            dst[pl.ds(m * 16, 16)] = dv[pl.ds(j * K + m * 16, 16)]
        pltpu.sync_copy(rows, acc.at[dst], add=True)

    def outer(o, carry):
        goff = s * TECW + o * GW
        pltpu.sync_copy(gidx.at[pl.ds(goff, GW)], gv)
        pltpu.sync_copy(dloc.at[pl.ds(c * CAPW + goff, GW)], dv)
        gather_start(0, rows0, sem0)

        def pair(p, carry2):
            j0 = 2 * p
            gather_start(j0 + 1, rows1, sem1)
            gather_wait(j0, rows0, sem0)
            scatter(j0, rows0, dst0)
            gather_start(j0 + 2, rows0, sem0)
            gather_wait(j0 + 1, rows1, sem1)
            scatter(j0 + 1, rows1, dst1)
            return carry2

        lax.fori_loop(0, (INNER - 1) // 2, pair, 0)
        gather_wait(INNER - 1, rows0, sem0)
        scatter(INNER - 1, rows0, dst0)
        return carry

    lax.fori_loop(0, NG, outer, 0)
    plsc.subcore_barrier()
    pltpu.sync_copy(acc.at[pl.ds(base_row, RP)],
                    out.at[pl.ds(c * ACC + base_row, RP)])


def _build_idx(gather_idx, dst_idx):
    gpad = jnp.concatenate([gather_idx, jnp.zeros((PAD,), jnp.int32)])
    dpad = jnp.concatenate([dst_idx, jnp.full((PAD,), 2 * HALF, jnp.int32)])
    locs = []
    for c in range(2):
        rel = dpad - c * HALF
        locs.append(jnp.where((rel >= 0) & (rel < HALF), rel, HALF))
    return gpad, jnp.concatenate(locs)


def _unpad(padded):
    return jnp.concatenate([padded[:HALF], padded[ACC:ACC + HALF]], axis=0)


def kernel(users, items, items_neg, edge_users, edge_items,
           user_embeds, item_embeds):
    eu = edge_users.astype(jnp.int32)
    ei = edge_items.astype(jnp.int32)

    u_deg = jnp.bincount(eu, length=NU)
    i_deg = jnp.bincount(ei, length=NI)
    u_norm = jnp.clip(u_deg, 1, None).astype(jnp.float32) ** -0.5
    i_norm = jnp.clip(i_deg, 1, None).astype(jnp.float32) ** -0.5

    gu, du = _build_idx(ei, eu)
    gi, di = _build_idx(eu, ei)
    zeros = jnp.zeros((RP, D), jnp.float32)

    ue = [user_embeds]
    ie = [item_embeds]
    for _ in range(LAYERS):
        nu = u_norm[:, None] * _unpad(_spmm(i_norm[:, None] * ie[-1],
                                            gu, du, zeros))
        ni = i_norm[:, None] * _unpad(_spmm(u_norm[:, None] * ue[-1],
                                            gi, di, zeros))
        ue.append(nu)
        ie.append(ni)

    final_u = sum(ue) / float(len(ue))
    final_i = sum(ie) / float(len(ie))

    u = final_u[users]
    it = final_i[items]
    it_neg = final_i[items_neg]
    pos = (u * it).sum(-1)
    neg = (u[:, None] * it_neg).sum(-1)
    return pos, neg


# speedup vs baseline: 4.0508x; 1.9542x over previous
import functools
import jax
import jax.numpy as jnp
from jax import lax
from jax.experimental import pallas as pl
from jax.experimental.pallas import tpu as pltpu
from jax.experimental.pallas import tpu_sc as plsc

NU = 50000
NI = 50000
E = 800000
D = 64
LAYERS = 3

HALF = 25000
ACC = 25088
RP = ACC // 16
K = 80
INNER = 25
GW = INNER * K
NG = -(-E // (16 * GW))
TECW = NG * GW
CAPW = 16 * TECW
PAD = CAPW - E

_mesh = plsc.VectorSubcoreMesh(core_axis_name="c", subcore_axis_name="s")


@functools.partial(
    pl.kernel,
    out_type=jax.ShapeDtypeStruct((2 * ACC, D), jnp.float32),
    mesh=_mesh,
    compiler_params=pltpu.CompilerParams(use_tc_tiling_on_sc=False),
    scratch_types=[
        pltpu.VMEM((GW,), jnp.int32),
        pltpu.VMEM((GW,), jnp.int32),
        pltpu.VMEM((K,), jnp.int32),
        pltpu.VMEM((K,), jnp.int32),
        pltpu.VMEM((K, D), jnp.float32),
        pltpu.VMEM((K, D), jnp.float32),
        pltpu.VMEM_SHARED((ACC, D), jnp.float32),
        pltpu.SemaphoreType.DMA,
        pltpu.SemaphoreType.DMA,
    ],
)
def _spmm(tab, gidx, dloc, zeros, out,
          gv, dv, dst0, dst1, rows0, rows1, acc, sem0, sem1):
    c = lax.axis_index("c")
    s = lax.axis_index("s")
    base_row = s * RP

    pltpu.sync_copy(zeros, acc.at[pl.ds(base_row, RP)])
    plsc.subcore_barrier()

    def gather_start(j, rows, sem):
        pltpu.async_copy(tab.at[gv.at[pl.ds(j * K, K)]], rows, sem)

    def gather_wait(j, rows, sem):
        pltpu.make_async_copy(tab.at[gv.at[pl.ds(j * K, K)]], rows,
                              sem).wait()

    def scatter(j, rows, dst):
        for m in range(K // 16):
            dst[pl.ds(m * 16, 16)] = dv[pl.ds(j * K + m * 16, 16)]
        pltpu.sync_copy(rows, acc.at[dst], add=True)

    def outer(o, carry):
        goff = s * TECW + o * GW
        pltpu.sync_copy(gidx.at[pl.ds(goff, GW)], gv)
        pltpu.sync_copy(dloc.at[pl.ds(c * CAPW + goff, GW)], dv)
        gather_start(0, rows0, sem0)

        def pair(p, carry2):
            j0 = 2 * p
            gather_start(j0 + 1, rows1, sem1)
            gather_wait(j0, rows0, sem0)
            scatter(j0, rows0, dst0)
            gather_start(j0 + 2, rows0, sem0)
            gather_wait(j0 + 1, rows1, sem1)
            scatter(j0 + 1, rows1, dst1)
            return carry2

        lax.fori_loop(0, (INNER - 1) // 2, pair, 0)
        gather_wait(INNER - 1, rows0, sem0)
        scatter(INNER - 1, rows0, dst0)
        return carry

    lax.fori_loop(0, NG, outer, 0)
    plsc.subcore_barrier()
    pltpu.sync_copy(acc.at[pl.ds(base_row, RP)],
                    out.at[pl.ds(c * ACC + base_row, RP)])


def _build_idx(gather_idx, dst_idx):
    gpad = jnp.concatenate([gather_idx, jnp.zeros((PAD,), jnp.int32)])
    dpad = jnp.concatenate([dst_idx, jnp.full((PAD,), 2 * HALF, jnp.int32)])
    locs = []
    for c in range(2):
        rel = dpad - c * HALF
        locs.append(jnp.where((rel >= 0) & (rel < HALF), rel, HALF))
    return gpad, jnp.concatenate(locs)


def _unpad(padded):
    return jnp.concatenate([padded[:HALF], padded[ACC:ACC + HALF]], axis=0)


def kernel(users, items, items_neg, edge_users, edge_items,
           user_embeds, item_embeds):
    eu = edge_users.astype(jnp.int32)
    ei = edge_items.astype(jnp.int32)

    u_deg = jnp.bincount(eu, length=NU)
    i_deg = jnp.bincount(ei, length=NI)
    u_norm = jnp.clip(u_deg, 1, None).astype(jnp.float32) ** -0.5
    i_norm = jnp.clip(i_deg, 1, None).astype(jnp.float32) ** -0.5

    gu, du = _build_idx(ei, eu)
    gi, di = _build_idx(eu, ei)
    zeros = jnp.zeros((RP, D), jnp.float32)

    ue = [user_embeds]
    ie = [item_embeds]
    for _ in range(LAYERS):
        nu = u_norm[:, None] * _unpad(_spmm(i_norm[:, None] * ie[-1],
                                            gu, du, zeros))
        ni = i_norm[:, None] * _unpad(_spmm(u_norm[:, None] * ue[-1],
                                            gi, di, zeros))
        ue.append(nu)
        ie.append(ni)

    final_u = sum(ue) / float(len(ue))
    final_i = sum(ie) / float(len(ie))

    u = final_u[users]
    it = final_i[items]
    it_neg = final_i[items_neg]
    pos = (u * it).sum(-1)
    neg = (u[:, None] * it_neg).sum(-1)
    return pos, neg
